# baseline (speedup 1.0000x reference)
import functools
import jax
import jax.numpy as jnp
from jax import lax
from jax.experimental import pallas as pl
from jax.experimental.pallas import tpu as pltpu
from jax.experimental.pallas import tpu_sc as plsc

N = 10000
E = 320000
D = 128
NT = 32
EPT = E // NT
CPT = D // NT
CH = 6400
NCHUNK = E // CH
NV = N // 16

_mesh = plsc.VectorSubcoreMesh(core_axis_name="c", subcore_axis_name="s")
_sc_params = pltpu.CompilerParams(needs_layout_passes=False)


@functools.partial(
    pl.kernel,
    out_type=(jax.ShapeDtypeStruct((E,), jnp.float32),
              jax.ShapeDtypeStruct((NT, N), jnp.float32),
              jax.ShapeDtypeStruct((E,), jnp.int32)),
    mesh=_mesh,
    compiler_params=_sc_params,
    scratch_types=[pltpu.VMEM((N,), jnp.float32),
                   pltpu.VMEM((N,), jnp.float32),
                   pltpu.VMEM((EPT,), jnp.int32),
                   pltpu.VMEM((EPT,), jnp.int32),
                   pltpu.VMEM((EPT,), jnp.float32),
                   pltpu.VMEM((N,), jnp.float32),
                   pltpu.VMEM((EPT,), jnp.int32)],
)
def _edge_scores(src_hbm, dst_hbm, el_hbm, er_hbm, ee_hbm, denp_hbm, pk_hbm,
                 el_v, er_v, src_v, dst_v, ee_v, den_v, pk_v):
    wid = lax.axis_index("s") * 2 + lax.axis_index("c")
    base = wid * EPT
    pltpu.sync_copy(el_hbm, el_v)
    pltpu.sync_copy(er_hbm, er_v)
    pltpu.sync_copy(src_hbm.at[pl.ds(base, EPT)], src_v)
    pltpu.sync_copy(dst_hbm.at[pl.ds(base, EPT)], dst_v)

    def _mx_el(i, m):
        return jnp.maximum(m, el_v[pl.ds(i * 16, 16)])

    def _mx_er(i, m):
        return jnp.maximum(m, er_v[pl.ds(i * 16, 16)])

    m_el = lax.fori_loop(1, NV, _mx_el, el_v[pl.ds(0, 16)])
    m_er = lax.fori_loop(1, NV, _mx_er, er_v[pl.ds(0, 16)])
    big_m = jnp.max(m_el) + jnp.max(m_er)

    def _zero(i, c):
        den_v[pl.ds(i * 16, 16)] = jnp.zeros((16,), jnp.float32)
        return c

    lax.fori_loop(0, NV, _zero, 0)

    @plsc.parallel_loop(0, EPT // 16, unroll=8)
    def _edge(j):
        sl = pl.ds(j * 16, 16)
        sv = src_v[sl]
        dv = dst_v[sl]
        x = plsc.load_gather(el_v, [sv]) + plsc.load_gather(er_v, [dv])
        e = jnp.maximum(x, 0.2 * x)
        ee = jnp.exp(e - big_m)
        ee_v[sl] = ee
        pk_v[sl] = (dv << 14) | sv
        plsc.addupdate_scatter(den_v, [dv], ee)

    pltpu.sync_copy(ee_v, ee_hbm.at[pl.ds(base, EPT)])
    pltpu.sync_copy(den_v, denp_hbm.at[wid])
    pltpu.sync_copy(pk_v, pk_hbm.at[pl.ds(base, EPT)])


@functools.partial(
    pl.kernel,
    out_type=jax.ShapeDtypeStruct((D * N,), jnp.float32),
    mesh=_mesh,
    compiler_params=_sc_params,
    scratch_types=[pltpu.VMEM((CPT * N,), jnp.float32),
                   pltpu.VMEM((CPT * N,), jnp.float32),
                   pltpu.VMEM((CH,), jnp.int32),
                   pltpu.VMEM((CH,), jnp.float32),
                   pltpu.VMEM((CH,), jnp.int32),
                   pltpu.VMEM((CH,), jnp.float32),
                   pltpu.SemaphoreType.DMA,
                   pltpu.SemaphoreType.DMA],
)
def _aggregate(zt_hbm, pk_hbm, ee_hbm, out_hbm,
               z_v, acc_v, pkb_a, eeb_a, pkb_b, eeb_b, sem_a, sem_b):
    wid = lax.axis_index("s") * 2 + lax.axis_index("c")
    cbase = wid * (CPT * N)

    def _issue(eb, pb, ebuf, sem):
        pltpu.async_copy(pk_hbm.at[pl.ds(eb, CH)], pb, sem)
        pltpu.async_copy(ee_hbm.at[pl.ds(eb, CH)], ebuf, sem)

    def _drain(pb, ebuf, sem):
        pltpu.make_async_copy(pk_hbm.at[pl.ds(0, CH)], pb, sem).wait()
        pltpu.make_async_copy(ee_hbm.at[pl.ds(0, CH)], ebuf, sem).wait()

    def _process(pb, ebuf):
        @plsc.parallel_loop(0, CH // 16, unroll=12)
        def _inner(j):
            sl = pl.ds(j * 16, 16)
            pk = pb[sl]
            sv = pk & 16383
            dv = lax.shift_right_logical(pk, 14)
            ev = ebuf[sl]
            for col in range(CPT):
                svo = sv + (col * N) if col else sv
                dvo = dv + (col * N) if col else dv
                g = plsc.load_gather(z_v, [svo])
                plsc.addupdate_scatter(acc_v, [dvo], g * ev)

    _issue(0, pkb_a, eeb_a, sem_a)
    pltpu.sync_copy(zt_hbm.at[pl.ds(cbase, CPT * N)], z_v)

    def _zero(i, c):
        acc_v[pl.ds(i * 16, 16)] = jnp.zeros((16,), jnp.float32)
        return c

    lax.fori_loop(0, CPT * N // 16, _zero, 0)

    def _pair(k, c):
        _issue((2 * k + 1) * CH, pkb_b, eeb_b, sem_b)
        _drain(pkb_a, eeb_a, sem_a)
        _process(pkb_a, eeb_a)

        @pl.when(2 * k + 2 < NCHUNK)
        def _():
            _issue((2 * k + 2) * CH, pkb_a, eeb_a, sem_a)

        _drain(pkb_b, eeb_b, sem_b)
        _process(pkb_b, eeb_b)
        return c

    lax.fori_loop(0, NCHUNK // 2, _pair, 0)
    pltpu.sync_copy(acc_v, out_hbm.at[pl.ds(cbase, CPT * N)])


BN = N


def _dense_first_body(wt_ref, al_ref, ar_ref, ht_ref, zt_ref, el_ref, er_ref):
    zt = jnp.dot(wt_ref[...], ht_ref[...], preferred_element_type=jnp.float32)
    zt_ref[...] = zt
    el_ref[...] = jnp.dot(al_ref[...], zt, preferred_element_type=jnp.float32)
    er_ref[...] = jnp.dot(ar_ref[...], zt, preferred_element_type=jnp.float32)


_dense_first = pl.pallas_call(
    _dense_first_body,
    grid=(N // BN,),
    in_specs=[pl.BlockSpec((D, D), lambda i: (0, 0)),
              pl.BlockSpec((1, D), lambda i: (0, 0)),
              pl.BlockSpec((1, D), lambda i: (0, 0)),
              pl.BlockSpec((D, BN), lambda i: (0, i))],
    out_specs=[pl.BlockSpec((D, BN), lambda i: (0, i)),
               pl.BlockSpec((1, BN), lambda i: (0, i)),
               pl.BlockSpec((1, BN), lambda i: (0, i))],
    out_shape=[jax.ShapeDtypeStruct((D, N), jnp.float32),
               jax.ShapeDtypeStruct((1, N), jnp.float32),
               jax.ShapeDtypeStruct((1, N), jnp.float32)],
)


def _dense_mid_body(wt_ref, al_ref, ar_ref, b_ref, outu_ref, denp_ref,
                    zt_ref, el_ref, er_ref):
    den = jnp.sum(denp_ref[...], axis=0, keepdims=True) + 1e-9
    h2 = jnp.tanh(outu_ref[...] / den + b_ref[...])
    zt = jnp.dot(wt_ref[...], h2, preferred_element_type=jnp.float32)
    zt_ref[...] = zt
    el_ref[...] = jnp.dot(al_ref[...], zt, preferred_element_type=jnp.float32)
    er_ref[...] = jnp.dot(ar_ref[...], zt, preferred_element_type=jnp.float32)


_dense_mid = pl.pallas_call(
    _dense_mid_body,
    grid=(N // BN,),
    in_specs=[pl.BlockSpec((D, D), lambda i: (0, 0)),
              pl.BlockSpec((1, D), lambda i: (0, 0)),
              pl.BlockSpec((1, D), lambda i: (0, 0)),
              pl.BlockSpec((D, 1), lambda i: (0, 0)),
              pl.BlockSpec((D, BN), lambda i: (0, i)),
              pl.BlockSpec((NT, BN), lambda i: (0, i))],
    out_specs=[pl.BlockSpec((D, BN), lambda i: (0, i)),
               pl.BlockSpec((1, BN), lambda i: (0, i)),
               pl.BlockSpec((1, BN), lambda i: (0, i))],
    out_shape=[jax.ShapeDtypeStruct((D, N), jnp.float32),
               jax.ShapeDtypeStruct((1, N), jnp.float32),
               jax.ShapeDtypeStruct((1, N), jnp.float32)],
)


def _final_body(sw1t_ref, sb1_ref, sw2r_ref, sb2_ref,
                outu0_ref, denp0_ref, b0_ref,
                outu1_ref, denp1_ref, b1_ref, r0_ref, r1_ref):
    den0 = jnp.sum(denp0_ref[...], axis=0, keepdims=True) + 1e-9
    z0 = jnp.tanh(outu0_ref[...] / den0 + b0_ref[...])
    den1 = jnp.sum(denp1_ref[...], axis=0, keepdims=True) + 1e-9
    z1 = jnp.tanh(outu1_ref[...] / den1 + b1_ref[...])
    q0 = jnp.maximum(
        jnp.dot(sw1t_ref[...], z0, preferred_element_type=jnp.float32)
        + sb1_ref[...], 0.0)
    q1 = jnp.maximum(
        jnp.dot(sw1t_ref[...], z1, preferred_element_type=jnp.float32)
        + sb1_ref[...], 0.0)
    w0 = jnp.dot(sw2r_ref[...], q0, preferred_element_type=jnp.float32) + sb2_ref[...]
    w1 = jnp.dot(sw2r_ref[...], q1, preferred_element_type=jnp.float32) + sb2_ref[...]
    m = jnp.maximum(w0, w1)
    a0 = jnp.exp(w0 - m)
    a1 = jnp.exp(w1 - m)
    s = a0 + a1
    r0_ref[...] = (a0 / s) * z0
    r1_ref[...] = (a1 / s) * z1


_final = pl.pallas_call(
    _final_body,
    grid=(N // BN,),
    in_specs=[pl.BlockSpec((D, D), lambda i: (0, 0)),
              pl.BlockSpec((D, 1), lambda i: (0, 0)),
              pl.BlockSpec((1, D), lambda i: (0, 0)),
              pl.BlockSpec((1, 1), lambda i: (0, 0)),
              pl.BlockSpec((D, BN), lambda i: (0, i)),
              pl.BlockSpec((NT, BN), lambda i: (0, i)),
              pl.BlockSpec((D, 1), lambda i: (0, 0)),
              pl.BlockSpec((D, BN), lambda i: (0, i)),
              pl.BlockSpec((NT, BN), lambda i: (0, i)),
              pl.BlockSpec((D, 1), lambda i: (0, 0))],
    out_specs=[pl.BlockSpec((D, BN), lambda i: (0, i)),
               pl.BlockSpec((D, BN), lambda i: (0, i))],
    out_shape=[jax.ShapeDtypeStruct((D, N), jnp.float32),
               jax.ShapeDtypeStruct((D, N), jnp.float32)],
)


def _gat_metapath(ht, src, dst, W1, al1, ar1, b1, W2, al2, ar2):
    zt1, el1, er1 = _dense_first(jnp.swapaxes(W1, 0, 1), al1.reshape(1, D),
                                 ar1.reshape(1, D), ht)
    ee1, denp1, pk1 = _edge_scores(src, dst, el1.reshape(N), er1.reshape(N))
    outu1 = _aggregate(zt1.reshape(D * N), pk1, ee1)
    zt2, el2, er2 = _dense_mid(jnp.swapaxes(W2, 0, 1), al2.reshape(1, D),
                               ar2.reshape(1, D), b1.reshape(D, 1),
                               outu1.reshape(D, N), denp1)
    ee2, denp2, pk2 = _edge_scores(src, dst, el2.reshape(N), er2.reshape(N))
    outu2 = _aggregate(zt2.reshape(D * N), pk2, ee2)
    return outu2.reshape(D, N), denp2


def kernel(h, edge_index0, edge_index1,
           W1_0, al1_0, ar1_0, b1_0, W2_0, al2_0, ar2_0, b2_0,
           W1_1, al1_1, ar1_1, b1_1, W2_1, al2_1, ar2_1, b2_1,
           sem_W1, sem_b1, sem_W2, sem_b2):
    ht = jnp.swapaxes(h, 0, 1)
    outu0, denp0 = _gat_metapath(ht, edge_index0[0], edge_index0[1],
                                 W1_0, al1_0, ar1_0, b1_0, W2_0, al2_0, ar2_0)
    outu1, denp1 = _gat_metapath(ht, edge_index1[0], edge_index1[1],
                                 W1_1, al1_1, ar1_1, b1_1, W2_1, al2_1, ar2_1)
    r0, r1 = _final(jnp.swapaxes(sem_W1, 0, 1), sem_b1.reshape(D, 1),
                    jnp.swapaxes(sem_W2, 0, 1), sem_b2.reshape(1, 1),
                    outu0, denp0, b2_0.reshape(D, 1),
                    outu1, denp1, b2_1.reshape(D, 1))
    return jnp.concatenate([jnp.swapaxes(r0, 0, 1), jnp.swapaxes(r1, 0, 1)], axis=1)


# speedup vs baseline: 1.0166x; 1.0166x over previous
import functools
import jax
import jax.numpy as jnp
from jax import lax
from jax.experimental import pallas as pl
from jax.experimental.pallas import tpu as pltpu
from jax.experimental.pallas import tpu_sc as plsc

N = 10000
E = 320000
D = 128
NT = 32
EPT = E // NT
CPT = D // NT
CH = 8000
NCHUNK = E // CH
NV = N // 16

_mesh = plsc.VectorSubcoreMesh(core_axis_name="c", subcore_axis_name="s")
_sc_params = pltpu.CompilerParams(needs_layout_passes=False)


@functools.partial(
    pl.kernel,
    out_type=(jax.ShapeDtypeStruct((E,), jnp.float32),
              jax.ShapeDtypeStruct((NT, N), jnp.float32),
              jax.ShapeDtypeStruct((E,), jnp.int32)),
    mesh=_mesh,
    compiler_params=_sc_params,
    scratch_types=[pltpu.VMEM((N,), jnp.float32),
                   pltpu.VMEM((N,), jnp.float32),
                   pltpu.VMEM((EPT,), jnp.int32),
                   pltpu.VMEM((EPT,), jnp.int32),
                   pltpu.VMEM((EPT,), jnp.float32),
                   pltpu.VMEM((N,), jnp.float32),
                   pltpu.VMEM((EPT,), jnp.int32)],
)
def _edge_scores(src_hbm, dst_hbm, el_hbm, er_hbm, ee_hbm, denp_hbm, pk_hbm,
                 el_v, er_v, src_v, dst_v, ee_v, den_v, pk_v):
    wid = lax.axis_index("s") * 2 + lax.axis_index("c")
    base = wid * EPT
    pltpu.sync_copy(el_hbm, el_v)
    pltpu.sync_copy(er_hbm, er_v)
    pltpu.sync_copy(src_hbm.at[pl.ds(base, EPT)], src_v)
    pltpu.sync_copy(dst_hbm.at[pl.ds(base, EPT)], dst_v)

    def _mx_el(i, m):
        return jnp.maximum(m, el_v[pl.ds(i * 16, 16)])

    def _mx_er(i, m):
        return jnp.maximum(m, er_v[pl.ds(i * 16, 16)])

    m_el = lax.fori_loop(1, NV, _mx_el, el_v[pl.ds(0, 16)])
    m_er = lax.fori_loop(1, NV, _mx_er, er_v[pl.ds(0, 16)])
    big_m = jnp.max(m_el) + jnp.max(m_er)

    def _zero(i, c):
        den_v[pl.ds(i * 16, 16)] = jnp.zeros((16,), jnp.float32)
        return c

    lax.fori_loop(0, NV, _zero, 0)

    @plsc.parallel_loop(0, EPT // 16, unroll=8)
    def _edge(j):
        sl = pl.ds(j * 16, 16)
        sv = src_v[sl]
        dv = dst_v[sl]
        x = plsc.load_gather(el_v, [sv]) + plsc.load_gather(er_v, [dv])
        e = jnp.maximum(x, 0.2 * x)
        ee = jnp.exp(e - big_m)
        ee_v[sl] = ee
        pk_v[sl] = (dv << 14) | sv
        plsc.addupdate_scatter(den_v, [dv], ee)

    pltpu.sync_copy(ee_v, ee_hbm.at[pl.ds(base, EPT)])
    pltpu.sync_copy(den_v, denp_hbm.at[wid])
    pltpu.sync_copy(pk_v, pk_hbm.at[pl.ds(base, EPT)])


@functools.partial(
    pl.kernel,
    out_type=jax.ShapeDtypeStruct((D * N,), jnp.float32),
    mesh=_mesh,
    compiler_params=_sc_params,
    scratch_types=[pltpu.VMEM((CPT * N,), jnp.float32),
                   pltpu.VMEM((CPT * N,), jnp.float32),
                   pltpu.VMEM((CH,), jnp.int32),
                   pltpu.VMEM((CH,), jnp.float32),
                   pltpu.VMEM((CH,), jnp.int32),
                   pltpu.VMEM((CH,), jnp.float32),
                   pltpu.SemaphoreType.DMA,
                   pltpu.SemaphoreType.DMA],
)
def _aggregate(zt_hbm, pk_hbm, ee_hbm, out_hbm,
               z_v, acc_v, pkb_a, eeb_a, pkb_b, eeb_b, sem_a, sem_b):
    wid = lax.axis_index("s") * 2 + lax.axis_index("c")
    cbase = wid * (CPT * N)

    def _issue(eb, pb, ebuf, sem):
        pltpu.async_copy(pk_hbm.at[pl.ds(eb, CH)], pb, sem)
        pltpu.async_copy(ee_hbm.at[pl.ds(eb, CH)], ebuf, sem)

    def _drain(pb, ebuf, sem):
        pltpu.make_async_copy(pk_hbm.at[pl.ds(0, CH)], pb, sem).wait()
        pltpu.make_async_copy(ee_hbm.at[pl.ds(0, CH)], ebuf, sem).wait()

    def _process(pb, ebuf):
        @plsc.parallel_loop(0, CH // 16, unroll=10)
        def _inner(j):
            sl = pl.ds(j * 16, 16)
            pk = pb[sl]
            sv = pk & 16383
            dv = lax.shift_right_logical(pk, 14)
            ev = ebuf[sl]
            for col in range(CPT):
                svo = sv + (col * N) if col else sv
                dvo = dv + (col * N) if col else dv
                g = plsc.load_gather(z_v, [svo])
                plsc.addupdate_scatter(acc_v, [dvo], g * ev)

    _issue(0, pkb_a, eeb_a, sem_a)
    pltpu.sync_copy(zt_hbm.at[pl.ds(cbase, CPT * N)], z_v)

    def _zero(i, c):
        acc_v[pl.ds(i * 16, 16)] = jnp.zeros((16,), jnp.float32)
        return c

    lax.fori_loop(0, CPT * N // 16, _zero, 0)

    def _pair(k, c):
        _issue((2 * k + 1) * CH, pkb_b, eeb_b, sem_b)
        _drain(pkb_a, eeb_a, sem_a)
        _process(pkb_a, eeb_a)

        @pl.when(2 * k + 2 < NCHUNK)
        def _():
            _issue((2 * k + 2) * CH, pkb_a, eeb_a, sem_a)

        _drain(pkb_b, eeb_b, sem_b)
        _process(pkb_b, eeb_b)
        return c

    lax.fori_loop(0, NCHUNK // 2, _pair, 0)
    pltpu.sync_copy(acc_v, out_hbm.at[pl.ds(cbase, CPT * N)])


BN = N


def _dense_first_body(wt_ref, al_ref, ar_ref, ht_ref, zt_ref, el_ref, er_ref):
    zt = jnp.dot(wt_ref[...], ht_ref[...], preferred_element_type=jnp.float32)
    zt_ref[...] = zt
    el_ref[...] = jnp.dot(al_ref[...], zt, preferred_element_type=jnp.float32)
    er_ref[...] = jnp.dot(ar_ref[...], zt, preferred_element_type=jnp.float32)


_dense_first = pl.pallas_call(
    _dense_first_body,
    grid=(N // BN,),
    in_specs=[pl.BlockSpec((D, D), lambda i: (0, 0)),
              pl.BlockSpec((1, D), lambda i: (0, 0)),
              pl.BlockSpec((1, D), lambda i: (0, 0)),
              pl.BlockSpec((D, BN), lambda i: (0, i))],
    out_specs=[pl.BlockSpec((D, BN), lambda i: (0, i)),
               pl.BlockSpec((1, BN), lambda i: (0, i)),
               pl.BlockSpec((1, BN), lambda i: (0, i))],
    out_shape=[jax.ShapeDtypeStruct((D, N), jnp.float32),
               jax.ShapeDtypeStruct((1, N), jnp.float32),
               jax.ShapeDtypeStruct((1, N), jnp.float32)],
)


def _dense_mid_body(wt_ref, al_ref, ar_ref, b_ref, outu_ref, denp_ref,
                    zt_ref, el_ref, er_ref):
    den = jnp.sum(denp_ref[...], axis=0, keepdims=True) + 1e-9
    h2 = jnp.tanh(outu_ref[...] / den + b_ref[...])
    zt = jnp.dot(wt_ref[...], h2, preferred_element_type=jnp.float32)
    zt_ref[...] = zt
    el_ref[...] = jnp.dot(al_ref[...], zt, preferred_element_type=jnp.float32)
    er_ref[...] = jnp.dot(ar_ref[...], zt, preferred_element_type=jnp.float32)


_dense_mid = pl.pallas_call(
    _dense_mid_body,
    grid=(N // BN,),
    in_specs=[pl.BlockSpec((D, D), lambda i: (0, 0)),
              pl.BlockSpec((1, D), lambda i: (0, 0)),
              pl.BlockSpec((1, D), lambda i: (0, 0)),
              pl.BlockSpec((D, 1), lambda i: (0, 0)),
              pl.BlockSpec((D, BN), lambda i: (0, i)),
              pl.BlockSpec((NT, BN), lambda i: (0, i))],
    out_specs=[pl.BlockSpec((D, BN), lambda i: (0, i)),
               pl.BlockSpec((1, BN), lambda i: (0, i)),
               pl.BlockSpec((1, BN), lambda i: (0, i))],
    out_shape=[jax.ShapeDtypeStruct((D, N), jnp.float32),
               jax.ShapeDtypeStruct((1, N), jnp.float32),
               jax.ShapeDtypeStruct((1, N), jnp.float32)],
)


def _final_body(sw1t_ref, sb1_ref, sw2r_ref, sb2_ref,
                outu0_ref, denp0_ref, b0_ref,
                outu1_ref, denp1_ref, b1_ref, r0_ref, r1_ref):
    den0 = jnp.sum(denp0_ref[...], axis=0, keepdims=True) + 1e-9
    z0 = jnp.tanh(outu0_ref[...] / den0 + b0_ref[...])
    den1 = jnp.sum(denp1_ref[...], axis=0, keepdims=True) + 1e-9
    z1 = jnp.tanh(outu1_ref[...] / den1 + b1_ref[...])
    q0 = jnp.maximum(
        jnp.dot(sw1t_ref[...], z0, preferred_element_type=jnp.float32)
        + sb1_ref[...], 0.0)
    q1 = jnp.maximum(
        jnp.dot(sw1t_ref[...], z1, preferred_element_type=jnp.float32)
        + sb1_ref[...], 0.0)
    w0 = jnp.dot(sw2r_ref[...], q0, preferred_element_type=jnp.float32) + sb2_ref[...]
    w1 = jnp.dot(sw2r_ref[...], q1, preferred_element_type=jnp.float32) + sb2_ref[...]
    m = jnp.maximum(w0, w1)
    a0 = jnp.exp(w0 - m)
    a1 = jnp.exp(w1 - m)
    s = a0 + a1
    r0_ref[...] = (a0 / s) * z0
    r1_ref[...] = (a1 / s) * z1


_final = pl.pallas_call(
    _final_body,
    grid=(N // BN,),
    in_specs=[pl.BlockSpec((D, D), lambda i: (0, 0)),
              pl.BlockSpec((D, 1), lambda i: (0, 0)),
              pl.BlockSpec((1, D), lambda i: (0, 0)),
              pl.BlockSpec((1, 1), lambda i: (0, 0)),
              pl.BlockSpec((D, BN), lambda i: (0, i)),
              pl.BlockSpec((NT, BN), lambda i: (0, i)),
              pl.BlockSpec((D, 1), lambda i: (0, 0)),
              pl.BlockSpec((D, BN), lambda i: (0, i)),
              pl.BlockSpec((NT, BN), lambda i: (0, i)),
              pl.BlockSpec((D, 1), lambda i: (0, 0))],
    out_specs=[pl.BlockSpec((D, BN), lambda i: (0, i)),
               pl.BlockSpec((D, BN), lambda i: (0, i))],
    out_shape=[jax.ShapeDtypeStruct((D, N), jnp.float32),
               jax.ShapeDtypeStruct((D, N), jnp.float32)],
)


def _gat_metapath(ht, src, dst, W1, al1, ar1, b1, W2, al2, ar2):
    zt1, el1, er1 = _dense_first(jnp.swapaxes(W1, 0, 1), al1.reshape(1, D),
                                 ar1.reshape(1, D), ht)
    ee1, denp1, pk1 = _edge_scores(src, dst, el1.reshape(N), er1.reshape(N))
    outu1 = _aggregate(zt1.reshape(D * N), pk1, ee1)
    zt2, el2, er2 = _dense_mid(jnp.swapaxes(W2, 0, 1), al2.reshape(1, D),
                               ar2.reshape(1, D), b1.reshape(D, 1),
                               outu1.reshape(D, N), denp1)
    ee2, denp2, pk2 = _edge_scores(src, dst, el2.reshape(N), er2.reshape(N))
    outu2 = _aggregate(zt2.reshape(D * N), pk2, ee2)
    return outu2.reshape(D, N), denp2


def kernel(h, edge_index0, edge_index1,
           W1_0, al1_0, ar1_0, b1_0, W2_0, al2_0, ar2_0, b2_0,
           W1_1, al1_1, ar1_1, b1_1, W2_1, al2_1, ar2_1, b2_1,
           sem_W1, sem_b1, sem_W2, sem_b2):
    ht = jnp.swapaxes(h, 0, 1)
    outu0, denp0 = _gat_metapath(ht, edge_index0[0], edge_index0[1],
                                 W1_0, al1_0, ar1_0, b1_0, W2_0, al2_0, ar2_0)
    outu1, denp1 = _gat_metapath(ht, edge_index1[0], edge_index1[1],
                                 W1_1, al1_1, ar1_1, b1_1, W2_1, al2_1, ar2_1)
    r0, r1 = _final(jnp.swapaxes(sem_W1, 0, 1), sem_b1.reshape(D, 1),
                    jnp.swapaxes(sem_W2, 0, 1), sem_b2.reshape(1, 1),
                    outu0, denp0, b2_0.reshape(D, 1),
                    outu1, denp1, b2_1.reshape(D, 1))
    return jnp.concatenate([jnp.swapaxes(r0, 0, 1), jnp.swapaxes(r1, 0, 1)], axis=1)


# speedup vs baseline: 1.0400x; 1.0231x over previous
import functools
import jax
import jax.numpy as jnp
from jax import lax
from jax.experimental import pallas as pl
from jax.experimental.pallas import tpu as pltpu
from jax.experimental.pallas import tpu_sc as plsc

N = 10000
E = 320000
D = 128
NT = 32
EPT = E // NT
CPT = D // NT
CH = 6400
NCHUNK = E // CH
NV = N // 16

_mesh = plsc.VectorSubcoreMesh(core_axis_name="c", subcore_axis_name="s")
_sc_params = pltpu.CompilerParams(needs_layout_passes=False)


@functools.partial(
    pl.kernel,
    out_type=(jax.ShapeDtypeStruct((E,), jnp.float32),
              jax.ShapeDtypeStruct((NT, N), jnp.float32),
              jax.ShapeDtypeStruct((E,), jnp.int32)),
    mesh=_mesh,
    compiler_params=_sc_params,
    scratch_types=[pltpu.VMEM((N,), jnp.float32),
                   pltpu.VMEM((N,), jnp.float32),
                   pltpu.VMEM((EPT,), jnp.int32),
                   pltpu.VMEM((EPT,), jnp.int32),
                   pltpu.VMEM((EPT,), jnp.float32),
                   pltpu.VMEM((N,), jnp.float32),
                   pltpu.VMEM((EPT,), jnp.int32),
                   pltpu.SemaphoreType.DMA],
)
def _edge_scores(src_hbm, dst_hbm, el_hbm, er_hbm, ee_hbm, denp_hbm, pk_hbm,
                 el_v, er_v, src_v, dst_v, ee_v, den_v, pk_v, sem):
    wid = lax.axis_index("s") * 2 + lax.axis_index("c")
    base = wid * EPT
    pltpu.async_copy(el_hbm, el_v, sem)
    pltpu.async_copy(er_hbm, er_v, sem)
    pltpu.async_copy(src_hbm.at[pl.ds(base, EPT)], src_v, sem)
    pltpu.async_copy(dst_hbm.at[pl.ds(base, EPT)], dst_v, sem)

    @plsc.parallel_loop(0, NV, unroll=8)
    def _zero(i):
        den_v[pl.ds(i * 16, 16)] = jnp.zeros((16,), jnp.float32)

    pltpu.make_async_copy(el_hbm, el_v, sem).wait()
    pltpu.make_async_copy(er_hbm, er_v, sem).wait()

    def _mx(i, ms):
        return (jnp.maximum(ms[0], el_v[pl.ds(i * 16, 16)]),
                jnp.maximum(ms[1], er_v[pl.ds(i * 16, 16)]))

    m_el, m_er = lax.fori_loop(1, NV, _mx,
                               (el_v[pl.ds(0, 16)], er_v[pl.ds(0, 16)]))
    big_m = jnp.max(m_el) + jnp.max(m_er)

    pltpu.make_async_copy(src_hbm.at[pl.ds(base, EPT)], src_v, sem).wait()
    pltpu.make_async_copy(dst_hbm.at[pl.ds(base, EPT)], dst_v, sem).wait()

    @plsc.parallel_loop(0, EPT // 16, unroll=8)
    def _edge(j):
        sl = pl.ds(j * 16, 16)
        sv = src_v[sl]
        dv = dst_v[sl]
        x = plsc.load_gather(el_v, [sv]) + plsc.load_gather(er_v, [dv])
        e = jnp.maximum(x, 0.2 * x)
        ee = jnp.exp(e - big_m)
        ee_v[sl] = ee
        pk_v[sl] = (dv << 14) | sv
        plsc.addupdate_scatter(den_v, [dv], ee)

    pltpu.sync_copy(ee_v, ee_hbm.at[pl.ds(base, EPT)])
    pltpu.sync_copy(den_v, denp_hbm.at[wid])
    pltpu.sync_copy(pk_v, pk_hbm.at[pl.ds(base, EPT)])


@functools.partial(
    pl.kernel,
    out_type=jax.ShapeDtypeStruct((D * N,), jnp.float32),
    mesh=_mesh,
    compiler_params=_sc_params,
    scratch_types=[pltpu.VMEM((CPT * N,), jnp.float32),
                   pltpu.VMEM((CPT * N,), jnp.float32),
                   pltpu.VMEM((CH,), jnp.int32),
                   pltpu.VMEM((CH,), jnp.float32),
                   pltpu.VMEM((CH,), jnp.int32),
                   pltpu.VMEM((CH,), jnp.float32),
                   pltpu.SemaphoreType.DMA,
                   pltpu.SemaphoreType.DMA],
)
def _aggregate(zt_hbm, pk_hbm, ee_hbm, out_hbm,
               z_v, acc_v, pkb_a, eeb_a, pkb_b, eeb_b, sem_a, sem_b):
    wid = lax.axis_index("s") * 2 + lax.axis_index("c")
    cbase = wid * (CPT * N)

    def _issue(eb, pb, ebuf, sem):
        pltpu.async_copy(pk_hbm.at[pl.ds(eb, CH)], pb, sem)
        pltpu.async_copy(ee_hbm.at[pl.ds(eb, CH)], ebuf, sem)

    def _drain(pb, ebuf, sem):
        pltpu.make_async_copy(pk_hbm.at[pl.ds(0, CH)], pb, sem).wait()
        pltpu.make_async_copy(ee_hbm.at[pl.ds(0, CH)], ebuf, sem).wait()

    def _process(pb, ebuf):
        @plsc.parallel_loop(0, CH // 16, unroll=10)
        def _inner(j):
            sl = pl.ds(j * 16, 16)
            pk = pb[sl]
            sv = pk & 16383
            dv = lax.shift_right_logical(pk, 14)
            ev = ebuf[sl]
            for col in range(CPT):
                svo = sv + (col * N) if col else sv
                dvo = dv + (col * N) if col else dv
                g = plsc.load_gather(z_v, [svo])
                plsc.addupdate_scatter(acc_v, [dvo], g * ev)

    _issue(0, pkb_a, eeb_a, sem_a)
    pltpu.sync_copy(zt_hbm.at[pl.ds(cbase, CPT * N)], z_v)

    def _zero(i, c):
        acc_v[pl.ds(i * 16, 16)] = jnp.zeros((16,), jnp.float32)
        return c

    lax.fori_loop(0, CPT * N // 16, _zero, 0)

    def _pair(k, c):
        _issue((2 * k + 1) * CH, pkb_b, eeb_b, sem_b)
        _drain(pkb_a, eeb_a, sem_a)
        _process(pkb_a, eeb_a)

        @pl.when(2 * k + 2 < NCHUNK)
        def _():
            _issue((2 * k + 2) * CH, pkb_a, eeb_a, sem_a)

        _drain(pkb_b, eeb_b, sem_b)
        _process(pkb_b, eeb_b)
        return c

    lax.fori_loop(0, NCHUNK // 2, _pair, 0)
    pltpu.sync_copy(acc_v, out_hbm.at[pl.ds(cbase, CPT * N)])


BN = N


def _dense_first_body(wt_ref, al_ref, ar_ref, ht_ref, zt_ref, el_ref, er_ref):
    zt = jnp.dot(wt_ref[...], ht_ref[...], preferred_element_type=jnp.float32)
    zt_ref[...] = zt
    el_ref[...] = jnp.dot(al_ref[...], zt, preferred_element_type=jnp.float32)
    er_ref[...] = jnp.dot(ar_ref[...], zt, preferred_element_type=jnp.float32)


_dense_first = pl.pallas_call(
    _dense_first_body,
    grid=(N // BN,),
    in_specs=[pl.BlockSpec((D, D), lambda i: (0, 0)),
              pl.BlockSpec((1, D), lambda i: (0, 0)),
              pl.BlockSpec((1, D), lambda i: (0, 0)),
              pl.BlockSpec((D, BN), lambda i: (0, i))],
    out_specs=[pl.BlockSpec((D, BN), lambda i: (0, i)),
               pl.BlockSpec((1, BN), lambda i: (0, i)),
               pl.BlockSpec((1, BN), lambda i: (0, i))],
    out_shape=[jax.ShapeDtypeStruct((D, N), jnp.float32),
               jax.ShapeDtypeStruct((1, N), jnp.float32),
               jax.ShapeDtypeStruct((1, N), jnp.float32)],
)


def _dense_mid_body(wt_ref, al_ref, ar_ref, b_ref, outu_ref, denp_ref,
                    zt_ref, el_ref, er_ref):
    den = jnp.sum(denp_ref[...], axis=0, keepdims=True) + 1e-9
    h2 = jnp.tanh(outu_ref[...] / den + b_ref[...])
    zt = jnp.dot(wt_ref[...], h2, preferred_element_type=jnp.float32)
    zt_ref[...] = zt
    el_ref[...] = jnp.dot(al_ref[...], zt, preferred_element_type=jnp.float32)
    er_ref[...] = jnp.dot(ar_ref[...], zt, preferred_element_type=jnp.float32)


_dense_mid = pl.pallas_call(
    _dense_mid_body,
    grid=(N // BN,),
    in_specs=[pl.BlockSpec((D, D), lambda i: (0, 0)),
              pl.BlockSpec((1, D), lambda i: (0, 0)),
              pl.BlockSpec((1, D), lambda i: (0, 0)),
              pl.BlockSpec((D, 1), lambda i: (0, 0)),
              pl.BlockSpec((D, BN), lambda i: (0, i)),
              pl.BlockSpec((NT, BN), lambda i: (0, i))],
    out_specs=[pl.BlockSpec((D, BN), lambda i: (0, i)),
               pl.BlockSpec((1, BN), lambda i: (0, i)),
               pl.BlockSpec((1, BN), lambda i: (0, i))],
    out_shape=[jax.ShapeDtypeStruct((D, N), jnp.float32),
               jax.ShapeDtypeStruct((1, N), jnp.float32),
               jax.ShapeDtypeStruct((1, N), jnp.float32)],
)


def _final_body(sw1t_ref, sb1_ref, sw2r_ref, sb2_ref,
                outu0_ref, denp0_ref, b0_ref,
                outu1_ref, denp1_ref, b1_ref, r0_ref, r1_ref):
    den0 = jnp.sum(denp0_ref[...], axis=0, keepdims=True) + 1e-9
    z0 = jnp.tanh(outu0_ref[...] / den0 + b0_ref[...])
    den1 = jnp.sum(denp1_ref[...], axis=0, keepdims=True) + 1e-9
    z1 = jnp.tanh(outu1_ref[...] / den1 + b1_ref[...])
    q0 = jnp.maximum(
        jnp.dot(sw1t_ref[...], z0, preferred_element_type=jnp.float32)
        + sb1_ref[...], 0.0)
    q1 = jnp.maximum(
        jnp.dot(sw1t_ref[...], z1, preferred_element_type=jnp.float32)
        + sb1_ref[...], 0.0)
    w0 = jnp.dot(sw2r_ref[...], q0, preferred_element_type=jnp.float32) + sb2_ref[...]
    w1 = jnp.dot(sw2r_ref[...], q1, preferred_element_type=jnp.float32) + sb2_ref[...]
    m = jnp.maximum(w0, w1)
    a0 = jnp.exp(w0 - m)
    a1 = jnp.exp(w1 - m)
    s = a0 + a1
    r0_ref[...] = (a0 / s) * z0
    r1_ref[...] = (a1 / s) * z1


_final = pl.pallas_call(
    _final_body,
    grid=(N // BN,),
    in_specs=[pl.BlockSpec((D, D), lambda i: (0, 0)),
              pl.BlockSpec((D, 1), lambda i: (0, 0)),
              pl.BlockSpec((1, D), lambda i: (0, 0)),
              pl.BlockSpec((1, 1), lambda i: (0, 0)),
              pl.BlockSpec((D, BN), lambda i: (0, i)),
              pl.BlockSpec((NT, BN), lambda i: (0, i)),
              pl.BlockSpec((D, 1), lambda i: (0, 0)),
              pl.BlockSpec((D, BN), lambda i: (0, i)),
              pl.BlockSpec((NT, BN), lambda i: (0, i)),
              pl.BlockSpec((D, 1), lambda i: (0, 0))],
    out_specs=[pl.BlockSpec((D, BN), lambda i: (0, i)),
               pl.BlockSpec((D, BN), lambda i: (0, i))],
    out_shape=[jax.ShapeDtypeStruct((D, N), jnp.float32),
               jax.ShapeDtypeStruct((D, N), jnp.float32)],
)


def _gat_metapath(ht, src, dst, W1, al1, ar1, b1, W2, al2, ar2):
    zt1, el1, er1 = _dense_first(jnp.swapaxes(W1, 0, 1), al1.reshape(1, D),
                                 ar1.reshape(1, D), ht)
    ee1, denp1, pk1 = _edge_scores(src, dst, el1.reshape(N), er1.reshape(N))
    outu1 = _aggregate(zt1.reshape(D * N), pk1, ee1)
    zt2, el2, er2 = _dense_mid(jnp.swapaxes(W2, 0, 1), al2.reshape(1, D),
                               ar2.reshape(1, D), b1.reshape(D, 1),
                               outu1.reshape(D, N), denp1)
    ee2, denp2, pk2 = _edge_scores(src, dst, el2.reshape(N), er2.reshape(N))
    outu2 = _aggregate(zt2.reshape(D * N), pk2, ee2)
    return outu2.reshape(D, N), denp2


def kernel(h, edge_index0, edge_index1,
           W1_0, al1_0, ar1_0, b1_0, W2_0, al2_0, ar2_0, b2_0,
           W1_1, al1_1, ar1_1, b1_1, W2_1, al2_1, ar2_1, b2_1,
           sem_W1, sem_b1, sem_W2, sem_b2):
    ht = jnp.swapaxes(h, 0, 1)
    outu0, denp0 = _gat_metapath(ht, edge_index0[0], edge_index0[1],
                                 W1_0, al1_0, ar1_0, b1_0, W2_0, al2_0, ar2_0)
    outu1, denp1 = _gat_metapath(ht, edge_index1[0], edge_index1[1],
                                 W1_1, al1_1, ar1_1, b1_1, W2_1, al2_1, ar2_1)
    r0, r1 = _final(jnp.swapaxes(sem_W1, 0, 1), sem_b1.reshape(D, 1),
                    jnp.swapaxes(sem_W2, 0, 1), sem_b2.reshape(1, 1),
                    outu0, denp0, b2_0.reshape(D, 1),
                    outu1, denp1, b2_1.reshape(D, 1))
    return jnp.concatenate([jnp.swapaxes(r0, 0, 1), jnp.swapaxes(r1, 0, 1)], axis=1)
